# baseline (speedup 1.0000x reference)
import functools

import jax
import jax.numpy as jnp
from jax import lax
from jax.experimental import pallas as pl
from jax.experimental.pallas import tpu as pltpu
from jax.experimental.pallas import tpu_sc as plsc

N = 10000
E = 320000
D = 128

NC = 2
NS = 16
NW = NC * NS

N_PAD = 10240
ROWS_PER_SUB = N_PAD // NS
K = 128
CHUNKS = 80
E_PAD = NW * CHUNKS * K
NBUF = 2

_mesh = plsc.VectorSubcoreMesh(core_axis_name="c", subcore_axis_name="s")


def _sc_degree_one(idx2d, zeros128, ones128):
    @functools.partial(
        pl.kernel,
        out_type=jax.ShapeDtypeStruct((NC, N_PAD, D), jnp.float32),
        mesh=_mesh,
        scratch_types=[
            pltpu.VMEM((CHUNKS, K), jnp.int32),
            pltpu.VMEM((K, D), jnp.float32),
            pltpu.VMEM_SHARED((N_PAD, D), jnp.float32),
            pltpu.SemaphoreType.DMA,
        ],
    )
    def k(idx_hbm, z_hbm, o_hbm, deg_hbm, idx_v, ones_v, acc_sh, sem):
        c = lax.axis_index("c")
        s = lax.axis_index("s")
        wid = s * NC + c
        row0 = s * ROWS_PER_SUB
        pltpu.sync_copy(z_hbm, acc_sh.at[pl.ds(row0, ROWS_PER_SUB)])
        pltpu.sync_copy(o_hbm, ones_v)
        pltpu.sync_copy(idx_hbm.at[wid], idx_v)
        plsc.subcore_barrier()

        @pl.loop(0, CHUNKS // 8)
        def _(r):
            for u in range(8):
                pltpu.async_copy(
                    ones_v, acc_sh.at[idx_v.at[r * 8 + u]], sem, add=True)
            for u in range(8):
                pltpu.make_async_copy(
                    ones_v, acc_sh.at[idx_v.at[r * 8 + u]], sem).wait()

        plsc.subcore_barrier()
        pltpu.sync_copy(acc_sh.at[pl.ds(row0, ROWS_PER_SUB)],
                        deg_hbm.at[c, pl.ds(row0, ROWS_PER_SUB)])

    return k(idx2d, zeros128, ones128)


KM = 80
E_PER_TILE = E // NW
CHUNKS_M = E_PER_TILE // KM


def _sc_msgpass(table, src, dst, zeros128):
    @functools.partial(
        pl.kernel,
        out_type=jax.ShapeDtypeStruct((NC, N_PAD, D), jnp.float32),
        mesh=_mesh,
        scratch_types=[
            pltpu.VMEM((KM,), jnp.int32),
            pltpu.VMEM((KM,), jnp.int32),
            pltpu.VMEM((KM,), jnp.int32),
            pltpu.VMEM((KM,), jnp.int32),
            pltpu.VMEM((KM, D), jnp.float32),
            pltpu.VMEM((KM, D), jnp.float32),
            pltpu.SemaphoreType.DMA,
            pltpu.SemaphoreType.DMA,
            pltpu.SemaphoreType.DMA,
            pltpu.SemaphoreType.DMA,
            pltpu.SemaphoreType.DMA,
            pltpu.SemaphoreType.DMA,
            pltpu.VMEM_SHARED((N_PAD, D), jnp.float32),
        ],
    )
    def k(t_hbm, src_hbm, dst_hbm, z_hbm, out_hbm,
          src_a, dst_a, src_b, dst_b, rows_a, rows_b,
          sas, sad, sbs, sbd, ssa, ssb, acc_sh):
        c = lax.axis_index("c")
        s = lax.axis_index("s")
        wid = s * NC + c
        row0 = s * ROWS_PER_SUB
        tile0 = wid * E_PER_TILE
        pltpu.sync_copy(src_hbm.at[pl.ds(tile0, KM)], src_a)
        pltpu.sync_copy(dst_hbm.at[pl.ds(tile0, KM)], dst_a)

        def fetch(buf_s, buf_d, sem_s, sem_d, base):
            pltpu.async_copy(src_hbm.at[pl.ds(base, KM)], buf_s, sem_s)
            pltpu.async_copy(dst_hbm.at[pl.ds(base, KM)], buf_d, sem_d)

        def wait_fetch(buf_s, buf_d, sem_s, sem_d, base):
            pltpu.make_async_copy(
                src_hbm.at[pl.ds(base, KM)], buf_s, sem_s).wait()
            pltpu.make_async_copy(
                dst_hbm.at[pl.ds(base, KM)], buf_d, sem_d).wait()

        pltpu.sync_copy(z_hbm, acc_sh.at[pl.ds(row0, ROWS_PER_SUB)])
        plsc.subcore_barrier()

        @pl.loop(0, CHUNKS_M // 2)
        def _(r):
            i = 2 * r
            fetch(src_b, dst_b, sbs, sbd, tile0 + (i + 1) * KM)
            pltpu.sync_copy(t_hbm.at[src_a], rows_a)
            pltpu.async_copy(rows_a, acc_sh.at[dst_a], ssa, add=True)
            wait_fetch(src_b, dst_b, sbs, sbd, tile0 + (i + 1) * KM)
            pltpu.sync_copy(t_hbm.at[src_b], rows_b)
            pltpu.make_async_copy(rows_a, acc_sh.at[dst_a], ssa).wait()
            fetch(src_a, dst_a, sas, sad, tile0 + (i + 2) * KM)
            sd = pltpu.async_copy(rows_b, acc_sh.at[dst_b], ssb, add=True)
            sd.wait()
            wait_fetch(src_a, dst_a, sas, sad, tile0 + (i + 2) * KM)

        pltpu.sync_copy(t_hbm.at[src_a], rows_a)
        pltpu.sync_copy(rows_a, acc_sh.at[dst_a], add=True)

        plsc.subcore_barrier()
        pltpu.sync_copy(acc_sh.at[pl.ds(row0, ROWS_PER_SUB)],
                        out_hbm.at[c, pl.ds(row0, ROWS_PER_SUB)])

    return k(table, src, dst, zeros128)


BR = 2000
NBLK = N // BR


def _norm_from_deg(dref):
    d0 = (dref[0] + dref[1])[:, 0:1]
    return jnp.where(d0 > 0.0, lax.rsqrt(jnp.maximum(d0, 1.0)), 0.0)


def _tc_prescale_body(x_ref, dgo_ref, o_ref):
    ns = _norm_from_deg(dgo_ref[...])
    o_ref[...] = x_ref[...] * ns


def _tc_prescale(x, dgo):
    return pl.pallas_call(
        _tc_prescale_body,
        grid=(NBLK,),
        in_specs=[
            pl.BlockSpec((BR, D), lambda i: (i, 0)),
            pl.BlockSpec((NC, BR, D), lambda i: (0, i, 0)),
        ],
        out_specs=pl.BlockSpec((BR, D), lambda i: (i, 0)),
        out_shape=jax.ShapeDtypeStruct((N_PAD, D), jnp.float32),
    )(x, dgo)


def _tc_mid_body(p_ref, dgi_ref, dgo_ref, w_ref, b_ref, o_ref):
    nd = _norm_from_deg(dgi_ref[...])
    agg = (p_ref[0] + p_ref[1]) * nd
    y = jnp.dot(agg, w_ref[...], preferred_element_type=jnp.float32)
    h = jnp.maximum(y + b_ref[...], 0.0)
    ns = _norm_from_deg(dgo_ref[...])
    o_ref[...] = h * ns


def _tc_mid(p, dgi, dgo, w, b):
    return pl.pallas_call(
        _tc_mid_body,
        grid=(NBLK,),
        in_specs=[
            pl.BlockSpec((NC, BR, D), lambda i: (0, i, 0)),
            pl.BlockSpec((NC, BR, D), lambda i: (0, i, 0)),
            pl.BlockSpec((NC, BR, D), lambda i: (0, i, 0)),
            pl.BlockSpec((D, D), lambda i: (0, 0)),
            pl.BlockSpec((1, D), lambda i: (0, 0)),
        ],
        out_specs=pl.BlockSpec((BR, D), lambda i: (i, 0)),
        out_shape=jax.ShapeDtypeStruct((N_PAD, D), jnp.float32),
    )(p, dgi, dgo, w, b)


def _tc_final_body(p_ref, dgi_ref, w_ref, b_ref, o_ref):
    nd = _norm_from_deg(dgi_ref[...])
    agg = (p_ref[0] + p_ref[1]) * nd
    y = jnp.dot(agg, w_ref[...], preferred_element_type=jnp.float32)
    o_ref[...] = jnp.maximum(y + b_ref[...], 0.0)


def _tc_final(p, dgi, w, b):
    return pl.pallas_call(
        _tc_final_body,
        grid=(NBLK,),
        in_specs=[
            pl.BlockSpec((NC, BR, D), lambda i: (0, i, 0)),
            pl.BlockSpec((NC, BR, D), lambda i: (0, i, 0)),
            pl.BlockSpec((D, D), lambda i: (0, 0)),
            pl.BlockSpec((1, D), lambda i: (0, 0)),
        ],
        out_specs=pl.BlockSpec((BR, D), lambda i: (i, 0)),
        out_shape=jax.ShapeDtypeStruct((N, D), jnp.float32),
    )(p, dgi, w, b)


def kernel(x, edge_index, W1, b1, W2, b2):
    pad = jnp.full((E_PAD - E,), N, jnp.int32)
    src2d = jnp.concatenate([edge_index[0], pad]).reshape(NW, CHUNKS, K)
    dst2d = jnp.concatenate([edge_index[1], pad]).reshape(NW, CHUNKS, K)
    ones128 = jnp.ones((K, D), jnp.float32)
    zeros128 = jnp.zeros((ROWS_PER_SUB, D), jnp.float32)
    b1r = b1.reshape(1, D)
    b2r = b2.reshape(1, D)

    dgo = _sc_degree_one(src2d, zeros128, ones128)
    t0 = _tc_prescale(x, dgo)
    dgi = _sc_degree_one(dst2d, zeros128, ones128)
    p1 = _sc_msgpass(t0, edge_index[0], edge_index[1], zeros128)
    t1 = _tc_mid(p1, dgi, dgo, W1, b1r)
    p2 = _sc_msgpass(t1, edge_index[0], edge_index[1], zeros128)
    out = _tc_final(p2, dgi, W2, b2r)
    return out


# speedup vs baseline: 1.1028x; 1.1028x over previous
import functools

import jax
import jax.numpy as jnp
from jax import lax
from jax.experimental import pallas as pl
from jax.experimental.pallas import tpu as pltpu
from jax.experimental.pallas import tpu_sc as plsc

N = 10000
E = 320000
D = 128

NC = 2
NS = 16
NW = NC * NS

N_PAD = 10240
ROWS_PER_SUB = N_PAD // NS
K = 128
CHUNKS = 80
E_PAD = NW * CHUNKS * K
NBUF = 2

_mesh = plsc.VectorSubcoreMesh(core_axis_name="c", subcore_axis_name="s")


def _sc_degree_one(idx2d, zeros128, ones128):
    @functools.partial(
        pl.kernel,
        out_type=jax.ShapeDtypeStruct((NC, N_PAD, D), jnp.float32),
        mesh=_mesh,
        scratch_types=[
            pltpu.VMEM((CHUNKS, K), jnp.int32),
            pltpu.VMEM((K, D), jnp.float32),
            pltpu.VMEM_SHARED((N_PAD, D), jnp.float32),
            pltpu.SemaphoreType.DMA,
        ],
    )
    def k(idx_hbm, z_hbm, o_hbm, deg_hbm, idx_v, ones_v, acc_sh, sem):
        c = lax.axis_index("c")
        s = lax.axis_index("s")
        wid = s * NC + c
        row0 = s * ROWS_PER_SUB
        pltpu.sync_copy(z_hbm, acc_sh.at[pl.ds(row0, ROWS_PER_SUB)])
        pltpu.sync_copy(o_hbm, ones_v)
        pltpu.sync_copy(idx_hbm.at[wid], idx_v)
        plsc.subcore_barrier()

        @pl.loop(0, CHUNKS // 8)
        def _(r):
            for u in range(8):
                pltpu.async_copy(
                    ones_v, acc_sh.at[idx_v.at[r * 8 + u]], sem, add=True)
            for u in range(8):
                pltpu.make_async_copy(
                    ones_v, acc_sh.at[idx_v.at[r * 8 + u]], sem).wait()

        plsc.subcore_barrier()
        pltpu.sync_copy(acc_sh.at[pl.ds(row0, ROWS_PER_SUB)],
                        deg_hbm.at[c, pl.ds(row0, ROWS_PER_SUB)])

    return k(idx2d, zeros128, ones128)


KM = 80
E_PER_TILE = E // NW
CHUNKS_M = E_PER_TILE // KM


def _sc_msgpass(table, src, dst, zeros128):
    @functools.partial(
        pl.kernel,
        out_type=jax.ShapeDtypeStruct((NC, N_PAD, D), jnp.float32),
        mesh=_mesh,
        scratch_types=(
            [pltpu.VMEM((KM,), jnp.int32)] * 8
            + [pltpu.VMEM((KM, D), jnp.float32)] * 2
            + [pltpu.SemaphoreType.DMA] * 10
            + [pltpu.VMEM_SHARED((N_PAD, D), jnp.float32)]
        ),
    )
    def k(t_hbm, src_hbm, dst_hbm, z_hbm, out_hbm,
          sv0, sv1, sv2, sv3, dv0, dv1, dv2, dv3, rw0, rw1,
          fs0, fs1, fs2, fs3, fd0, fd1, fd2, fd3, sc0, sc1, acc_sh):
        srcs = [sv0, sv1, sv2, sv3]
        dsts = [dv0, dv1, dv2, dv3]
        fsem = [(fs0, fd0), (fs1, fd1), (fs2, fd2), (fs3, fd3)]
        rows = [rw0, rw1]
        ssem = [sc0, sc1]
        c = lax.axis_index("c")
        s = lax.axis_index("s")
        wid = s * NC + c
        row0 = s * ROWS_PER_SUB
        tile0 = wid * E_PER_TILE

        def fetch(j, base):
            pltpu.async_copy(src_hbm.at[pl.ds(base, KM)], srcs[j], fsem[j][0])
            pltpu.async_copy(dst_hbm.at[pl.ds(base, KM)], dsts[j], fsem[j][1])

        def wait_fetch(j, base):
            pltpu.make_async_copy(
                src_hbm.at[pl.ds(base, KM)], srcs[j], fsem[j][0]).wait()
            pltpu.make_async_copy(
                dst_hbm.at[pl.ds(base, KM)], dsts[j], fsem[j][1]).wait()

        def wait_scatter(b, j):
            pltpu.make_async_copy(
                rows[b], acc_sh.at[dsts[j]], ssem[b]).wait()

        for j in range(2):
            fetch(j, tile0 + j * KM)
        pltpu.sync_copy(z_hbm, acc_sh.at[pl.ds(row0, ROWS_PER_SUB)])
        plsc.subcore_barrier()

        @pl.loop(0, (CHUNKS_M - 1) // 4)
        def _(r):
            i0 = 4 * r
            for j in range(4):
                b = j % 2
                i = i0 + j
                wait_fetch(j, tile0 + i * KM)
                if j < 2:
                    @pl.when(r > 0)
                    def _():
                        wait_scatter(b, (j + 2) % 4)
                else:
                    wait_scatter(b, (j + 2) % 4)
                pltpu.sync_copy(t_hbm.at[srcs[j]], rows[b])
                pltpu.async_copy(rows[b], acc_sh.at[dsts[j]], ssem[b],
                                 add=True)
                nj = (j + 2) % 4
                if j == 3:
                    @pl.when(r < (CHUNKS_M - 1) // 4 - 1)
                    def _():
                        fetch(nj, tile0 + (i + 2) * KM)
                else:
                    fetch(nj, tile0 + (i + 2) * KM)

        last = CHUNKS_M - 1
        wait_fetch(0, tile0 + last * KM)
        wait_scatter(0, 2)
        pltpu.sync_copy(t_hbm.at[srcs[0]], rows[0])
        pltpu.sync_copy(rows[0], acc_sh.at[dsts[0]], add=True)
        wait_scatter(1, 3)

        plsc.subcore_barrier()
        pltpu.sync_copy(acc_sh.at[pl.ds(row0, ROWS_PER_SUB)],
                        out_hbm.at[c, pl.ds(row0, ROWS_PER_SUB)])

    return k(table, src, dst, zeros128)


BR = 2000
NBLK = N // BR


def _norm_from_deg(dref):
    d0 = (dref[0] + dref[1])[:, 0:1]
    return jnp.where(d0 > 0.0, lax.rsqrt(jnp.maximum(d0, 1.0)), 0.0)


def _tc_prescale_body(x_ref, dgo_ref, o_ref):
    ns = _norm_from_deg(dgo_ref[...])
    o_ref[...] = x_ref[...] * ns


def _tc_prescale(x, dgo):
    return pl.pallas_call(
        _tc_prescale_body,
        grid=(NBLK,),
        in_specs=[
            pl.BlockSpec((BR, D), lambda i: (i, 0)),
            pl.BlockSpec((NC, BR, D), lambda i: (0, i, 0)),
        ],
        out_specs=pl.BlockSpec((BR, D), lambda i: (i, 0)),
        out_shape=jax.ShapeDtypeStruct((N_PAD, D), jnp.float32),
    )(x, dgo)


def _tc_mid_body(p_ref, dgi_ref, dgo_ref, w_ref, b_ref, o_ref):
    nd = _norm_from_deg(dgi_ref[...])
    agg = (p_ref[0] + p_ref[1]) * nd
    y = jnp.dot(agg, w_ref[...], preferred_element_type=jnp.float32)
    h = jnp.maximum(y + b_ref[...], 0.0)
    ns = _norm_from_deg(dgo_ref[...])
    o_ref[...] = h * ns


def _tc_mid(p, dgi, dgo, w, b):
    return pl.pallas_call(
        _tc_mid_body,
        grid=(NBLK,),
        in_specs=[
            pl.BlockSpec((NC, BR, D), lambda i: (0, i, 0)),
            pl.BlockSpec((NC, BR, D), lambda i: (0, i, 0)),
            pl.BlockSpec((NC, BR, D), lambda i: (0, i, 0)),
            pl.BlockSpec((D, D), lambda i: (0, 0)),
            pl.BlockSpec((1, D), lambda i: (0, 0)),
        ],
        out_specs=pl.BlockSpec((BR, D), lambda i: (i, 0)),
        out_shape=jax.ShapeDtypeStruct((N_PAD, D), jnp.float32),
    )(p, dgi, dgo, w, b)


def _tc_final_body(p_ref, dgi_ref, w_ref, b_ref, o_ref):
    nd = _norm_from_deg(dgi_ref[...])
    agg = (p_ref[0] + p_ref[1]) * nd
    y = jnp.dot(agg, w_ref[...], preferred_element_type=jnp.float32)
    o_ref[...] = jnp.maximum(y + b_ref[...], 0.0)


def _tc_final(p, dgi, w, b):
    return pl.pallas_call(
        _tc_final_body,
        grid=(NBLK,),
        in_specs=[
            pl.BlockSpec((NC, BR, D), lambda i: (0, i, 0)),
            pl.BlockSpec((NC, BR, D), lambda i: (0, i, 0)),
            pl.BlockSpec((D, D), lambda i: (0, 0)),
            pl.BlockSpec((1, D), lambda i: (0, 0)),
        ],
        out_specs=pl.BlockSpec((BR, D), lambda i: (i, 0)),
        out_shape=jax.ShapeDtypeStruct((N, D), jnp.float32),
    )(p, dgi, w, b)


def kernel(x, edge_index, W1, b1, W2, b2):
    pad = jnp.full((E_PAD - E,), N, jnp.int32)
    src2d = jnp.concatenate([edge_index[0], pad]).reshape(NW, CHUNKS, K)
    dst2d = jnp.concatenate([edge_index[1], pad]).reshape(NW, CHUNKS, K)
    ones128 = jnp.ones((K, D), jnp.float32)
    zeros128 = jnp.zeros((ROWS_PER_SUB, D), jnp.float32)
    b1r = b1.reshape(1, D)
    b2r = b2.reshape(1, D)

    dgo = _sc_degree_one(src2d, zeros128, ones128)
    t0 = _tc_prescale(x, dgo)
    dgi = _sc_degree_one(dst2d, zeros128, ones128)
    p1 = _sc_msgpass(t0, edge_index[0], edge_index[1], zeros128)
    t1 = _tc_mid(p1, dgi, dgo, W1, b1r)
    p2 = _sc_msgpass(t1, edge_index[0], edge_index[1], zeros128)
    out = _tc_final(p2, dgi, W2, b2r)
    return out
